# Initial kernel scaffold; baseline (speedup 1.0000x reference)
import functools

import jax
import jax.numpy as jnp
from jax import lax
from jax.experimental import pallas as pl
from jax.experimental.pallas import tpu as pltpu
from jax.experimental.pallas import tpu_sc as plsc

NC = 2
NS = 16
NW = NC * NS
CH = 128


@functools.cache
def _build(B, D_NL, D_CODE, D_AST, nch):
    mesh = plsc.VectorSubcoreMesh(core_axis_name="c", subcore_axis_name="s")
    bpw = B // NW

    @functools.partial(
        pl.kernel,
        out_type=(
            jax.ShapeDtypeStruct((B, D_NL), jnp.float32),
            jax.ShapeDtypeStruct((B, D_CODE), jnp.float32),
            jax.ShapeDtypeStruct((B, D_AST), jnp.float32),
        ),
        mesh=mesh,
        scratch_types=[
            pltpu.VMEM((nch, CH), jnp.int32),
            pltpu.VMEM((nch, CH), jnp.int32),
            pltpu.VMEM((nch, CH), jnp.int32),
            pltpu.VMEM((CH, D_NL), jnp.float32),
            pltpu.VMEM((CH, D_CODE), jnp.float32),
            pltpu.VMEM((CH, D_AST), jnp.float32),
            pltpu.SemaphoreType.DMA,
            pltpu.SemaphoreType.DMA,
        ],
    )
    def k(nl_ids, code_ids, ast_ids, nl_table, code_table, ast_table,
          nl_out, code_out, ast_out,
          nl_idx_v, code_idx_v, ast_idx_v, nl_rows, code_rows, ast_rows,
          gsem, ssem):
        wid = lax.axis_index("s") * NC + lax.axis_index("c")
        base = wid * bpw
        pltpu.sync_copy(nl_ids.at[wid], nl_idx_v)
        pltpu.sync_copy(code_ids.at[wid], code_idx_v)
        pltpu.sync_copy(ast_ids.at[wid], ast_idx_v)

        @pl.loop(0, nch)
        def _chunk(c):
            g1 = pltpu.async_copy(nl_table.at[nl_idx_v.at[c]], nl_rows, gsem)
            g2 = pltpu.async_copy(code_table.at[code_idx_v.at[c]], code_rows, gsem)
            g3 = pltpu.async_copy(ast_table.at[ast_idx_v.at[c]], ast_rows, gsem)
            g1.wait()
            g2.wait()
            g3.wait()
            off = base + c * CH
            s1 = pltpu.async_copy(nl_rows, nl_out.at[pl.ds(off, CH)], ssem)
            s2 = pltpu.async_copy(code_rows, code_out.at[pl.ds(off, CH)], ssem)
            s3 = pltpu.async_copy(ast_rows, ast_out.at[pl.ds(off, CH)], ssem)
            s1.wait()
            s2.wait()
            s3.wait()

    return k


def kernel(nl_token_ids, code_token_ids, ast_token_ids,
           nl_table, code_table, ast_table):
    Bt, S = nl_token_ids.shape
    B = Bt * S
    assert B % (NW * CH) == 0
    nch = B // (NW * CH)
    k = _build(B, nl_table.shape[1], code_table.shape[1], ast_table.shape[1], nch)
    ids3 = [x.reshape(NW, nch, CH) for x in
            (nl_token_ids, code_token_ids, ast_token_ids)]
    nl_out, code_out, ast_out = k(*ids3, nl_table, code_table, ast_table)
    return (nl_out.reshape(Bt, S, -1),
            code_out.reshape(Bt, S, -1),
            ast_out.reshape(Bt, S, -1))


# speedup vs baseline: 6.5715x; 6.5715x over previous
import functools

import jax
import jax.numpy as jnp
from jax import lax
from jax.experimental import pallas as pl
from jax.experimental.pallas import tpu as pltpu
from jax.experimental.pallas import tpu_sc as plsc

NC = 2
NS = 16
NW = NC * NS
CH = 128


@functools.cache
def _build(B, D_NL, D_CODE, D_AST, nch):
    mesh = plsc.VectorSubcoreMesh(core_axis_name="c", subcore_axis_name="s")
    bpw = B // NW

    @functools.partial(
        pl.kernel,
        out_type=(
            jax.ShapeDtypeStruct((B, D_NL), jnp.float32),
            jax.ShapeDtypeStruct((B, D_CODE), jnp.float32),
            jax.ShapeDtypeStruct((B, D_AST), jnp.float32),
        ),
        mesh=mesh,
        compiler_params=pltpu.CompilerParams(use_tc_tiling_on_sc=False),
        scratch_types=[
            pltpu.VMEM((nch, CH), jnp.int32),
            pltpu.VMEM((nch, CH), jnp.int32),
            pltpu.VMEM((nch, CH), jnp.int32),
            pltpu.VMEM((CH, D_NL), jnp.float32),
            pltpu.VMEM((CH, D_CODE), jnp.float32),
            pltpu.VMEM((CH, D_AST), jnp.float32),
            pltpu.SemaphoreType.DMA,
            pltpu.SemaphoreType.DMA,
        ],
    )
    def k(nl_ids, code_ids, ast_ids, nl_table, code_table, ast_table,
          nl_out, code_out, ast_out,
          nl_idx_v, code_idx_v, ast_idx_v, nl_rows, code_rows, ast_rows,
          gsem, ssem):
        wid = lax.axis_index("s") * NC + lax.axis_index("c")
        base = wid * bpw
        pltpu.sync_copy(nl_ids.at[wid], nl_idx_v)
        pltpu.sync_copy(code_ids.at[wid], code_idx_v)
        pltpu.sync_copy(ast_ids.at[wid], ast_idx_v)

        @pl.loop(0, nch)
        def _chunk(c):
            g1 = pltpu.async_copy(nl_table.at[nl_idx_v.at[c]], nl_rows, gsem)
            g2 = pltpu.async_copy(code_table.at[code_idx_v.at[c]], code_rows, gsem)
            g3 = pltpu.async_copy(ast_table.at[ast_idx_v.at[c]], ast_rows, gsem)
            g1.wait()
            g2.wait()
            g3.wait()
            off = base + c * CH
            s1 = pltpu.async_copy(nl_rows, nl_out.at[pl.ds(off, CH)], ssem)
            s2 = pltpu.async_copy(code_rows, code_out.at[pl.ds(off, CH)], ssem)
            s3 = pltpu.async_copy(ast_rows, ast_out.at[pl.ds(off, CH)], ssem)
            s1.wait()
            s2.wait()
            s3.wait()

    return k


def kernel(nl_token_ids, code_token_ids, ast_token_ids,
           nl_table, code_table, ast_table):
    Bt, S = nl_token_ids.shape
    B = Bt * S
    assert B % (NW * CH) == 0
    nch = B // (NW * CH)
    k = _build(B, nl_table.shape[1], code_table.shape[1], ast_table.shape[1], nch)
    ids3 = [x.reshape(NW, nch, CH) for x in
            (nl_token_ids, code_token_ids, ast_token_ids)]
    nl_out, code_out, ast_out = k(*ids3, nl_table, code_table, ast_table)
    return (nl_out.reshape(Bt, S, -1),
            code_out.reshape(Bt, S, -1),
            ast_out.reshape(Bt, S, -1))


# speedup vs baseline: 7.1176x; 1.0831x over previous
import functools

import jax
import jax.numpy as jnp
from jax import lax
from jax.experimental import pallas as pl
from jax.experimental.pallas import tpu as pltpu
from jax.experimental.pallas import tpu_sc as plsc

NC = 2
NS = 16
NW = NC * NS
CH = 64


@functools.cache
def _build(B, D_NL, D_CODE, D_AST, nch):
    mesh = plsc.VectorSubcoreMesh(core_axis_name="c", subcore_axis_name="s")
    bpw = B // NW
    npair = nch // 2

    @functools.partial(
        pl.kernel,
        out_type=(
            jax.ShapeDtypeStruct((B, D_NL), jnp.float32),
            jax.ShapeDtypeStruct((B, D_CODE), jnp.float32),
            jax.ShapeDtypeStruct((B, D_AST), jnp.float32),
        ),
        mesh=mesh,
        compiler_params=pltpu.CompilerParams(use_tc_tiling_on_sc=False),
        scratch_types=[
            pltpu.VMEM((nch, CH), jnp.int32),
            pltpu.VMEM((nch, CH), jnp.int32),
            pltpu.VMEM((nch, CH), jnp.int32),
            pltpu.VMEM((2, CH, D_NL), jnp.float32),
            pltpu.VMEM((2, CH, D_CODE), jnp.float32),
            pltpu.VMEM((2, CH, D_AST), jnp.float32),
            pltpu.SemaphoreType.DMA,
            pltpu.SemaphoreType.DMA,
            pltpu.SemaphoreType.DMA,
            pltpu.SemaphoreType.DMA,
        ],
    )
    def k(nl_ids, code_ids, ast_ids, nl_table, code_table, ast_table,
          nl_out, code_out, ast_out,
          nl_idx_v, code_idx_v, ast_idx_v, nl_rows, code_rows, ast_rows,
          gsem0, gsem1, ssem0, ssem1):
        wid = lax.axis_index("s") * NC + lax.axis_index("c")
        base = wid * bpw
        pltpu.sync_copy(nl_ids.at[wid], nl_idx_v)
        pltpu.sync_copy(code_ids.at[wid], code_idx_v)
        pltpu.sync_copy(ast_ids.at[wid], ast_idx_v)

        def gather(c, slot, gsem):
            pltpu.async_copy(nl_table.at[nl_idx_v.at[c]], nl_rows.at[slot], gsem)
            pltpu.async_copy(code_table.at[code_idx_v.at[c]], code_rows.at[slot], gsem)
            pltpu.async_copy(ast_table.at[ast_idx_v.at[c]], ast_rows.at[slot], gsem)

        def gather_wait(c, slot, gsem):
            pltpu.make_async_copy(nl_table.at[nl_idx_v.at[c]], nl_rows.at[slot], gsem).wait()
            pltpu.make_async_copy(code_table.at[code_idx_v.at[c]], code_rows.at[slot], gsem).wait()
            pltpu.make_async_copy(ast_table.at[ast_idx_v.at[c]], ast_rows.at[slot], gsem).wait()

        def store(c, slot, ssem):
            off = base + c * CH
            pltpu.async_copy(nl_rows.at[slot], nl_out.at[pl.ds(off, CH)], ssem)
            pltpu.async_copy(code_rows.at[slot], code_out.at[pl.ds(off, CH)], ssem)
            pltpu.async_copy(ast_rows.at[slot], ast_out.at[pl.ds(off, CH)], ssem)

        def store_wait(c, slot, ssem):
            off = base + c * CH
            pltpu.make_async_copy(nl_rows.at[slot], nl_out.at[pl.ds(off, CH)], ssem).wait()
            pltpu.make_async_copy(code_rows.at[slot], code_out.at[pl.ds(off, CH)], ssem).wait()
            pltpu.make_async_copy(ast_rows.at[slot], ast_out.at[pl.ds(off, CH)], ssem).wait()

        gather(0, 0, gsem0)

        @pl.loop(0, npair)
        def _pair(t):
            c0 = 2 * t
            c1 = c0 + 1

            @pl.when(t > 0)
            def _():
                store_wait(c0 - 1, 1, ssem1)
            gather(c1, 1, gsem1)

            gather_wait(c0, 0, gsem0)
            store(c0, 0, ssem0)

            store_wait(c0, 0, ssem0)

            @pl.when(t + 1 < npair)
            def _():
                gather(c0 + 2, 0, gsem0)

            gather_wait(c1, 1, gsem1)
            store(c1, 1, ssem1)

        store_wait(nch - 1, 1, ssem1)

    return k


def kernel(nl_token_ids, code_token_ids, ast_token_ids,
           nl_table, code_table, ast_table):
    Bt, S = nl_token_ids.shape
    B = Bt * S
    assert B % (NW * CH * 2) == 0
    nch = B // (NW * CH)
    k = _build(B, nl_table.shape[1], code_table.shape[1], ast_table.shape[1], nch)
    ids3 = [x.reshape(NW, nch, CH) for x in
            (nl_token_ids, code_token_ids, ast_token_ids)]
    nl_out, code_out, ast_out = k(*ids3, nl_table, code_table, ast_table)
    return (nl_out.reshape(Bt, S, -1),
            code_out.reshape(Bt, S, -1),
            ast_out.reshape(Bt, S, -1))


# speedup vs baseline: 7.1708x; 1.0075x over previous
import functools

import jax
import jax.numpy as jnp
from jax import lax
from jax.experimental import pallas as pl
from jax.experimental.pallas import tpu as pltpu
from jax.experimental.pallas import tpu_sc as plsc

NC = 2
NS = 16
NW = NC * NS
CH = 64
NBUF = 4


@functools.cache
def _build(B, D_NL, D_CODE, D_AST, nch):
    mesh = plsc.VectorSubcoreMesh(core_axis_name="c", subcore_axis_name="s")
    bpw = B // NW
    nr = nch // NBUF

    @functools.partial(
        pl.kernel,
        out_type=(
            jax.ShapeDtypeStruct((B, D_NL), jnp.float32),
            jax.ShapeDtypeStruct((B, D_CODE), jnp.float32),
            jax.ShapeDtypeStruct((B, D_AST), jnp.float32),
        ),
        mesh=mesh,
        compiler_params=pltpu.CompilerParams(use_tc_tiling_on_sc=False),
        scratch_types=[
            pltpu.VMEM((NBUF, CH), jnp.int32),
            pltpu.VMEM((NBUF, CH), jnp.int32),
            pltpu.VMEM((NBUF, CH), jnp.int32),
            pltpu.VMEM((NBUF, CH, D_NL), jnp.float32),
            pltpu.VMEM((NBUF, CH, D_CODE), jnp.float32),
            pltpu.VMEM((NBUF, CH, D_AST), jnp.float32),
        ] + [pltpu.SemaphoreType.DMA] * (3 * NBUF),
    )
    def k(nl_ids, code_ids, ast_ids, nl_table, code_table, ast_table,
          nl_out, code_out, ast_out,
          nl_idx_v, code_idx_v, ast_idx_v, nl_rows, code_rows, ast_rows,
          *sems):
        gsem = sems[0:NBUF]
        ssem = sems[NBUF:2 * NBUF]
        isem = sems[2 * NBUF:3 * NBUF]
        wid = lax.axis_index("s") * NC + lax.axis_index("c")
        base = wid * bpw

        def idx_issue(c, s):
            pltpu.async_copy(nl_ids.at[wid, c], nl_idx_v.at[s], isem[s])
            pltpu.async_copy(code_ids.at[wid, c], code_idx_v.at[s], isem[s])
            pltpu.async_copy(ast_ids.at[wid, c], ast_idx_v.at[s], isem[s])

        def idx_wait(c, s):
            pltpu.make_async_copy(nl_ids.at[wid, c], nl_idx_v.at[s], isem[s]).wait()
            pltpu.make_async_copy(code_ids.at[wid, c], code_idx_v.at[s], isem[s]).wait()
            pltpu.make_async_copy(ast_ids.at[wid, c], ast_idx_v.at[s], isem[s]).wait()

        def gather_issue(s):
            pltpu.async_copy(nl_table.at[nl_idx_v.at[s]], nl_rows.at[s], gsem[s])
            pltpu.async_copy(code_table.at[code_idx_v.at[s]], code_rows.at[s], gsem[s])
            pltpu.async_copy(ast_table.at[ast_idx_v.at[s]], ast_rows.at[s], gsem[s])

        def gather_wait(s):
            pltpu.make_async_copy(nl_table.at[nl_idx_v.at[s]], nl_rows.at[s], gsem[s]).wait()
            pltpu.make_async_copy(code_table.at[code_idx_v.at[s]], code_rows.at[s], gsem[s]).wait()
            pltpu.make_async_copy(ast_table.at[ast_idx_v.at[s]], ast_rows.at[s], gsem[s]).wait()

        def store_issue(c, s):
            off = base + c * CH
            pltpu.async_copy(nl_rows.at[s], nl_out.at[pl.ds(off, CH)], ssem[s])
            pltpu.async_copy(code_rows.at[s], code_out.at[pl.ds(off, CH)], ssem[s])
            pltpu.async_copy(ast_rows.at[s], ast_out.at[pl.ds(off, CH)], ssem[s])

        def store_wait(c, s):
            off = base + c * CH
            pltpu.make_async_copy(nl_rows.at[s], nl_out.at[pl.ds(off, CH)], ssem[s]).wait()
            pltpu.make_async_copy(code_rows.at[s], code_out.at[pl.ds(off, CH)], ssem[s]).wait()
            pltpu.make_async_copy(ast_rows.at[s], ast_out.at[pl.ds(off, CH)], ssem[s]).wait()

        for s in range(NBUF):
            idx_issue(s, s)
        idx_wait(0, 0)
        gather_issue(0)
        idx_wait(1, 1)
        gather_issue(1)

        @pl.loop(0, nr)
        def _round(r):
            cbase = r * NBUF
            for s in range(NBUF):
                c = cbase + s
                gather_wait(s)
                store_issue(c, s)

                @pl.when(r < nr - 1)
                def _():
                    idx_issue(c + NBUF, s)

                s2 = (s + 2) % NBUF
                if s < 2:
                    @pl.when(r > 0)
                    def _():
                        store_wait(c - 2, s2)
                    idx_wait(c + 2, s2)
                    gather_issue(s2)
                else:
                    store_wait(c - 2, s - 2)

                    @pl.when(r < nr - 1)
                    def _():
                        idx_wait(c + 2, s2)
                        gather_issue(s2)

        store_wait(nch - 2, NBUF - 2)
        store_wait(nch - 1, NBUF - 1)

    return k


def kernel(nl_token_ids, code_token_ids, ast_token_ids,
           nl_table, code_table, ast_table):
    Bt, S = nl_token_ids.shape
    B = Bt * S
    assert B % (NW * CH * NBUF) == 0
    nch = B // (NW * CH)
    k = _build(B, nl_table.shape[1], code_table.shape[1], ast_table.shape[1], nch)
    ids3 = [x.reshape(NW, nch, CH) for x in
            (nl_token_ids, code_token_ids, ast_token_ids)]
    nl_out, code_out, ast_out = k(*ids3, nl_table, code_table, ast_table)
    return (nl_out.reshape(Bt, S, -1),
            code_out.reshape(Bt, S, -1),
            ast_out.reshape(Bt, S, -1))
